# baseline (speedup 1.0000x reference)
import jax
import jax.numpy as jnp
from jax import lax
from jax.experimental import pallas as pl
from jax.experimental.pallas import tpu as pltpu, tpu_sc as plsc

_V = 30522
_D = 768
_H = 256
_B = 256
_E = 3
_OUT = 3
_PAD = 128
_NW = 32
_GG = _B // _NW
_GE = _E * _B // _NW


def _sc_gather_body(emb_g_hbm, emb_e_hbm, idx_hbm,
                    out_g_hbm, out_e_hbm,
                    idx_v, rowsg_v, rowse_v, semg, seme):
    wid = lax.axis_index("s") * 2 + lax.axis_index("c")
    pltpu.sync_copy(idx_hbm.at[pl.ds(wid * _GG, _GG)], idx_v)
    cp_g = pltpu.async_copy(emb_g_hbm.at[idx_v], rowsg_v, semg)
    cps = [pltpu.async_copy(emb_e_hbm.at[i].at[idx_v], rowse_v.at[i], seme)
           for i in range(_E)]
    cp_g.wait()
    pltpu.sync_copy(rowsg_v, out_g_hbm.at[pl.ds(wid * _GG, _GG)])
    for i in range(_E):
        cps[i].wait()
        pltpu.sync_copy(rowse_v.at[i], out_e_hbm.at[i, pl.ds(wid * _GG, _GG)])


def _make_sc_gather():
    return pl.kernel(
        _sc_gather_body,
        out_type=(jax.ShapeDtypeStruct((_B, _D), jnp.float32),
                  jax.ShapeDtypeStruct((_E, _B, _D), jnp.float32)),
        mesh=plsc.VectorSubcoreMesh(core_axis_name="c", subcore_axis_name="s"),
        scratch_types=[
            pltpu.VMEM((_GG,), jnp.int32),
            pltpu.VMEM((_GG, _D), jnp.float32),
            pltpu.VMEM((_E, _GG, _D), jnp.float32),
            pltpu.SemaphoreType.DMA,
            pltpu.SemaphoreType.DMA,
        ],
    )


def _ln_rows(x, gamma, beta):
    mu = jnp.mean(x, axis=-1, keepdims=True)
    v = jnp.mean((x - mu) ** 2, axis=-1, keepdims=True)
    return (x - mu) * lax.rsqrt(v + 1e-5) * gamma + beta


def _tc_body(xg_ref, xe_ref, m0_ref, Wg_ref, bg_ref, We_ref, be_ref,
             Wg1_ref, bg1_ref, gga_ref, gbe_ref, Wg2_ref, bg2_ref,
             Wc1_ref, bc1_ref, cga_ref, cbe_ref, Wc2_ref, bc2_ref,
             out_c_ref, out_p_ref, acc_ref, ws_ref):
    f32 = jnp.float32
    i = pl.program_id(0)

    @pl.when(i == 0)
    def _gating():
        m0 = m0_ref[...]
        xg = xg_ref[...] * m0
        h = jax.nn.gelu(jnp.dot(xg, Wg_ref[...], preferred_element_type=f32)
                        + bg_ref[...].reshape(1, _D))
        g = (jnp.dot(h, Wg1_ref[...], preferred_element_type=f32)
             + bg1_ref[...].reshape(1, _H))
        g = jax.nn.gelu(_ln_rows(g, gga_ref[...].reshape(1, _H),
                                 gbe_ref[...].reshape(1, _H)))
        logits = (jnp.dot(g, Wg2_ref[...], preferred_element_type=f32)
                  + bg2_ref[...].reshape(1, _OUT))
        lmax = jnp.max(logits, axis=-1, keepdims=True)
        e = jnp.exp(logits - lmax)
        p = e / jnp.sum(e, axis=-1, keepdims=True)
        out_p_ref[...] = p
        p0, p1, p2 = p[:, 0:1], p[:, 1:2], p[:, 2:3]
        excl2 = (p2 <= p0) & (p2 <= p1)
        excl1 = jnp.logical_not(excl2) & (p1 <= p0) & (p1 < p2)
        excl0 = jnp.logical_not(excl2) & jnp.logical_not(excl1)
        w0 = jnp.where(excl0, 0.0, p0)
        w1 = jnp.where(excl1, 0.0, p1)
        w2 = jnp.where(excl2, 0.0, p2)
        denom = w0 + w1 + w2
        ws_ref[...] = jnp.concatenate(
            [w0 / denom, w1 / denom, w2 / denom], axis=1)
        acc_ref[...] = jnp.zeros((_B, _D), f32)

    @pl.when((i >= 1) & (i <= _E))
    def _expert():
        xe = xe_ref[0] * m0_ref[...]
        he = jax.nn.gelu(jnp.dot(xe, We_ref[0], preferred_element_type=f32)
                         + be_ref[0])
        ws = ws_ref[...]
        w = jnp.where(i == 1, ws[:, 0:1],
                      jnp.where(i == 2, ws[:, 1:2], ws[:, 2:3]))
        acc_ref[...] = acc_ref[...] + he * w

    @pl.when(i == _E + 1)
    def _classifier():
        c = (jnp.dot(acc_ref[...], Wc1_ref[...], preferred_element_type=f32)
             + bc1_ref[...].reshape(1, _H))
        c = jax.nn.gelu(_ln_rows(c, cga_ref[...].reshape(1, _H),
                                 cbe_ref[...].reshape(1, _H)))
        out_c_ref[...] = (jnp.dot(c, Wc2_ref[...],
                                  preferred_element_type=f32)
                          + bc2_ref[...].reshape(1, _OUT))


def _c1(i):
    return (0,)


def _c2(i):
    return (0, 0)


def _eix3(i):
    return (jnp.clip(i - 1, 0, _E - 1), 0, 0)


def _eix2(i):
    return (jnp.clip(i - 1, 0, _E - 1), 0)


_TC_GRID = (_E + 2,)
_TC_IN_SPECS = [
    pl.BlockSpec((_B, _D), _c2),
    pl.BlockSpec((1, _B, _D), _eix3),
    pl.BlockSpec((_B, 1), _c2),
    pl.BlockSpec((_D, _D), _c2),
    pl.BlockSpec((_D,), _c1),
    pl.BlockSpec((1, _D, _D), _eix3),
    pl.BlockSpec((1, 1, _D), _eix3),
    pl.BlockSpec((_D, _H), _c2),
    pl.BlockSpec((_H,), _c1),
    pl.BlockSpec((_H,), _c1),
    pl.BlockSpec((_H,), _c1),
    pl.BlockSpec((_H, _OUT), _c2),
    pl.BlockSpec((_OUT,), _c1),
    pl.BlockSpec((_D, _H), _c2),
    pl.BlockSpec((_H,), _c1),
    pl.BlockSpec((_H,), _c1),
    pl.BlockSpec((_H,), _c1),
    pl.BlockSpec((_H, _OUT), _c2),
    pl.BlockSpec((_OUT,), _c1),
]
_TC_OUT_SPECS = (pl.BlockSpec((_B, _OUT), _c2),
                 pl.BlockSpec((_B, _OUT), _c2))
_TC_SCRATCH = [pltpu.VMEM((_B, _D), jnp.float32),
               pltpu.VMEM((_B, _OUT), jnp.float32)]

_tc_dense = pl.pallas_call(
    _tc_body,
    grid=_TC_GRID,
    in_specs=_TC_IN_SPECS,
    out_specs=_TC_OUT_SPECS,
    scratch_shapes=_TC_SCRATCH,
    out_shape=(jax.ShapeDtypeStruct((_B, _OUT), jnp.float32),
               jax.ShapeDtypeStruct((_B, _OUT), jnp.float32)),
    compiler_params=pltpu.CompilerParams(
        dimension_semantics=("arbitrary",)),
)


def kernel(input_ids, attention_mask, emb_g, W_g, b_g, emb_e, W_e, b_e,
           Wg1, bg1, g_gamma, g_beta, Wg2, bg2,
           Wc1, bc1, c_gamma, c_beta, Wc2, bc2):
    ids0 = input_ids[:, 0]
    m0 = attention_mask[:, 0].astype(jnp.float32).reshape(_B, 1)
    x_g, xe3 = _make_sc_gather()(emb_g, emb_e, ids0)

    out_c, out_p = _tc_dense(
        x_g, xe3, m0, W_g, b_g, W_e, b_e.reshape(_E, 1, _D),
        Wg1, bg1, g_gamma, g_beta, Wg2, bg2,
        Wc1, bc1, c_gamma, c_beta, Wc2, bc2)
    return out_c, out_p


# speedup vs baseline: 1.0321x; 1.0321x over previous
import jax
import jax.numpy as jnp
from jax import lax
from jax.experimental import pallas as pl
from jax.experimental.pallas import tpu as pltpu, tpu_sc as plsc

_V = 30522
_D = 768
_H = 256
_B = 256
_E = 3
_OUT = 3
_PAD = 128
_NW = 32
_GG = _B // _NW
_GE = _E * _B // _NW


def _sc_gather_body(emb_g_hbm, emb_e_hbm, idx_hbm,
                    out_g_hbm, out_e_hbm,
                    idx_v, rowsg_v, rowse_v, semg, seme):
    wid = lax.axis_index("s") * 2 + lax.axis_index("c")
    pltpu.sync_copy(idx_hbm.at[pl.ds(wid * _GG, _GG)], idx_v)
    cp_g = pltpu.async_copy(emb_g_hbm.at[idx_v], rowsg_v, semg)
    cps = [pltpu.async_copy(emb_e_hbm.at[i].at[idx_v], rowse_v.at[i], seme)
           for i in range(_E)]
    cp_g.wait()
    pltpu.sync_copy(rowsg_v, out_g_hbm.at[pl.ds(wid * _GG, _GG)])
    for i in range(_E):
        cps[i].wait()
        pltpu.sync_copy(rowse_v.at[i], out_e_hbm.at[i, pl.ds(wid * _GG, _GG)])


def _make_sc_gather():
    return pl.kernel(
        _sc_gather_body,
        out_type=(jax.ShapeDtypeStruct((_B, _D), jnp.float32),
                  jax.ShapeDtypeStruct((_E, _B, _D), jnp.float32)),
        mesh=plsc.VectorSubcoreMesh(core_axis_name="c", subcore_axis_name="s"),
        scratch_types=[
            pltpu.VMEM((_GG,), jnp.int32),
            pltpu.VMEM((_GG, _D), jnp.float32),
            pltpu.VMEM((_E, _GG, _D), jnp.float32),
            pltpu.SemaphoreType.DMA,
            pltpu.SemaphoreType.DMA,
        ],
    )


def _ln_rows(x, gamma, beta):
    mu = jnp.mean(x, axis=-1, keepdims=True)
    v = jnp.mean((x - mu) ** 2, axis=-1, keepdims=True)
    return (x - mu) * lax.rsqrt(v + 1e-5) * gamma + beta


def _tc_body(xg_ref, xe_ref, m0_ref, Wg_ref, bg_ref, We_ref, be_ref,
             Wg1_ref, bg1_ref, gga_ref, gbe_ref, Wg2_ref, bg2_ref,
             Wc1_ref, bc1_ref, cga_ref, cbe_ref, Wc2_ref, bc2_ref,
             out_c_ref, out_p_ref, We_v, xe_v, sem_w, sem_x):
    f32 = jnp.float32
    cp_w = pltpu.make_async_copy(We_ref, We_v, sem_w)
    cp_w.start()
    cp_x = pltpu.make_async_copy(xe_ref, xe_v, sem_x)
    cp_x.start()
    m0 = m0_ref[...]
    xg = xg_ref[...] * m0
    h = jax.nn.gelu(jnp.dot(xg, Wg_ref[...], preferred_element_type=f32)
                    + bg_ref[...].reshape(1, _D))
    g = (jnp.dot(h, Wg1_ref[...], preferred_element_type=f32)
         + bg1_ref[...].reshape(1, _H))
    g = jax.nn.gelu(_ln_rows(g, gga_ref[...].reshape(1, _H),
                             gbe_ref[...].reshape(1, _H)))
    logits = (jnp.dot(g, Wg2_ref[...], preferred_element_type=f32)
              + bg2_ref[...].reshape(1, _OUT))
    lmax = jnp.max(logits, axis=-1, keepdims=True)
    e = jnp.exp(logits - lmax)
    p = e / jnp.sum(e, axis=-1, keepdims=True)
    out_p_ref[...] = p
    p0, p1, p2 = p[:, 0:1], p[:, 1:2], p[:, 2:3]
    excl2 = (p2 <= p0) & (p2 <= p1)
    excl1 = jnp.logical_not(excl2) & (p1 <= p0) & (p1 < p2)
    excl0 = jnp.logical_not(excl2) & jnp.logical_not(excl1)
    w0 = jnp.where(excl0, 0.0, p0)
    w1 = jnp.where(excl1, 0.0, p1)
    w2 = jnp.where(excl2, 0.0, p2)
    denom = w0 + w1 + w2
    ws = (w0 / denom, w1 / denom, w2 / denom)
    cp_x.wait()
    cp_w.wait()
    acc = jnp.zeros((_B, _D), f32)
    for i in range(_E):
        xe = xe_v[i] * m0
        he = jax.nn.gelu(jnp.dot(xe, We_v[i], preferred_element_type=f32)
                         + be_ref[i].reshape(1, _D))
        acc = acc + he * ws[i]
    c = (jnp.dot(acc, Wc1_ref[...], preferred_element_type=f32)
         + bc1_ref[...].reshape(1, _H))
    c = jax.nn.gelu(_ln_rows(c, cga_ref[...].reshape(1, _H),
                             cbe_ref[...].reshape(1, _H)))
    out_c_ref[...] = (jnp.dot(c, Wc2_ref[...], preferred_element_type=f32)
                      + bc2_ref[...].reshape(1, _OUT))


_TC_IN_SPECS = [
    pl.BlockSpec(memory_space=pltpu.MemorySpace.VMEM),
    pl.BlockSpec(memory_space=pl.ANY),
    pl.BlockSpec(memory_space=pltpu.MemorySpace.VMEM),
    pl.BlockSpec(memory_space=pltpu.MemorySpace.VMEM),
    pl.BlockSpec(memory_space=pltpu.MemorySpace.VMEM),
    pl.BlockSpec(memory_space=pl.ANY),
    pl.BlockSpec(memory_space=pltpu.MemorySpace.VMEM),
] + [pl.BlockSpec(memory_space=pltpu.MemorySpace.VMEM)] * 12
_TC_OUT_SPECS = (pl.BlockSpec(memory_space=pltpu.MemorySpace.VMEM),
                 pl.BlockSpec(memory_space=pltpu.MemorySpace.VMEM))
_TC_SCRATCH = [pltpu.VMEM((_E, _D, _D), jnp.float32),
               pltpu.VMEM((_E, _B, _D), jnp.float32),
               pltpu.SemaphoreType.DMA,
               pltpu.SemaphoreType.DMA]

_tc_dense = pl.pallas_call(
    _tc_body,
    in_specs=_TC_IN_SPECS,
    out_specs=_TC_OUT_SPECS,
    scratch_shapes=_TC_SCRATCH,
    out_shape=(jax.ShapeDtypeStruct((_B, _OUT), jnp.float32),
               jax.ShapeDtypeStruct((_B, _OUT), jnp.float32)),
)


def kernel(input_ids, attention_mask, emb_g, W_g, b_g, emb_e, W_e, b_e,
           Wg1, bg1, g_gamma, g_beta, Wg2, bg2,
           Wc1, bc1, c_gamma, c_beta, Wc2, bc2):
    ids0 = input_ids[:, 0]
    m0 = attention_mask[:, 0].astype(jnp.float32).reshape(_B, 1)
    x_g, xe3 = _make_sc_gather()(emb_g, emb_e, ids0)

    out_c, out_p = _tc_dense(
        x_g, xe3, m0, W_g, b_g, W_e, b_e,
        Wg1, bg1, g_gamma, g_beta, Wg2, bg2,
        Wc1, bc1, c_gamma, c_beta, Wc2, bc2)
    return out_c, out_p


# speedup vs baseline: 1.0387x; 1.0064x over previous
import jax
import jax.numpy as jnp
from jax import lax
from jax.experimental import pallas as pl
from jax.experimental.pallas import tpu as pltpu, tpu_sc as plsc

_V = 30522
_D = 768
_H = 256
_B = 256
_S = 128
_E = 3
_OUT = 3
_NS = 16
_TG = _B // _NS


def _sc_gather_body(ids_hbm, emb_g_hbm, emb_e_hbm,
                    out_g_hbm, out_e_hbm,
                    blk_v, rows0_v, rows1_v, sem0, sem1):
    c = lax.axis_index("c")
    s = lax.axis_index("s")
    base = s * _TG
    pltpu.sync_copy(ids_hbm.at[pl.ds(base, _TG)], blk_v)

    @pl.when(c == 0)
    def _():
        cp0 = pltpu.async_copy(emb_g_hbm.at[blk_v], rows0_v, sem0)
        cp1 = pltpu.async_copy(emb_e_hbm.at[0].at[blk_v], rows1_v, sem1)
        cp0.wait()
        pltpu.sync_copy(rows0_v, out_g_hbm.at[pl.ds(base, _TG)])
        cp1.wait()
        pltpu.sync_copy(rows1_v, out_e_hbm.at[0, pl.ds(base, _TG)])

    @pl.when(c == 1)
    def _():
        cp0 = pltpu.async_copy(emb_e_hbm.at[1].at[blk_v], rows0_v, sem0)
        cp1 = pltpu.async_copy(emb_e_hbm.at[2].at[blk_v], rows1_v, sem1)
        cp0.wait()
        pltpu.sync_copy(rows0_v, out_e_hbm.at[1, pl.ds(base, _TG)])
        cp1.wait()
        pltpu.sync_copy(rows1_v, out_e_hbm.at[2, pl.ds(base, _TG)])


def _make_sc_gather():
    return pl.kernel(
        _sc_gather_body,
        out_type=(jax.ShapeDtypeStruct((_B, _D), jnp.float32),
                  jax.ShapeDtypeStruct((_E, _B, _D), jnp.float32)),
        mesh=plsc.VectorSubcoreMesh(core_axis_name="c", subcore_axis_name="s"),
        scratch_types=[
            pltpu.VMEM((_TG,), jnp.int32),
            pltpu.VMEM((_TG, _D), jnp.float32),
            pltpu.VMEM((_TG, _D), jnp.float32),
            pltpu.SemaphoreType.DMA,
            pltpu.SemaphoreType.DMA,
        ],
    )


def _ln_rows(x, gamma, beta):
    mu = jnp.mean(x, axis=-1, keepdims=True)
    v = jnp.mean((x - mu) ** 2, axis=-1, keepdims=True)
    return (x - mu) * lax.rsqrt(v + 1e-5) * gamma + beta


def _tc_body(xg_ref, xe_ref, m0_ref, Wg_ref, bg_ref, We_ref, be_ref,
             Wg1_ref, bg1_ref, gga_ref, gbe_ref, Wg2_ref, bg2_ref,
             Wc1_ref, bc1_ref, cga_ref, cbe_ref, Wc2_ref, bc2_ref,
             out_c_ref, out_p_ref):
    f32 = jnp.float32
    m0 = m0_ref[...]
    xg = xg_ref[...] * m0
    h = jax.nn.gelu(jnp.dot(xg, Wg_ref[...], preferred_element_type=f32)
                    + bg_ref[...].reshape(1, _D))
    g = (jnp.dot(h, Wg1_ref[...], preferred_element_type=f32)
         + bg1_ref[...].reshape(1, _H))
    g = jax.nn.gelu(_ln_rows(g, gga_ref[...].reshape(1, _H),
                             gbe_ref[...].reshape(1, _H)))
    logits = (jnp.dot(g, Wg2_ref[...], preferred_element_type=f32)
              + bg2_ref[...].reshape(1, _OUT))
    lmax = jnp.max(logits, axis=-1, keepdims=True)
    e = jnp.exp(logits - lmax)
    p = e / jnp.sum(e, axis=-1, keepdims=True)
    out_p_ref[...] = p
    p0, p1, p2 = p[:, 0:1], p[:, 1:2], p[:, 2:3]
    excl2 = (p2 <= p0) & (p2 <= p1)
    excl1 = jnp.logical_not(excl2) & (p1 <= p0) & (p1 < p2)
    excl0 = jnp.logical_not(excl2) & jnp.logical_not(excl1)
    w0 = jnp.where(excl0, 0.0, p0)
    w1 = jnp.where(excl1, 0.0, p1)
    w2 = jnp.where(excl2, 0.0, p2)
    denom = w0 + w1 + w2
    ws = (w0 / denom, w1 / denom, w2 / denom)
    acc = jnp.zeros((_B, _D), f32)
    for i in range(_E):
        xe = xe_ref[i] * m0
        he = jax.nn.gelu(jnp.dot(xe, We_ref[i], preferred_element_type=f32)
                         + be_ref[i].reshape(1, _D))
        acc = acc + he * ws[i]
    c = (jnp.dot(acc, Wc1_ref[...], preferred_element_type=f32)
         + bc1_ref[...].reshape(1, _H))
    c = jax.nn.gelu(_ln_rows(c, cga_ref[...].reshape(1, _H),
                             cbe_ref[...].reshape(1, _H)))
    out_c_ref[...] = (jnp.dot(c, Wc2_ref[...], preferred_element_type=f32)
                      + bc2_ref[...].reshape(1, _OUT))


_tc_dense = pl.pallas_call(
    _tc_body,
    out_shape=(jax.ShapeDtypeStruct((_B, _OUT), jnp.float32),
               jax.ShapeDtypeStruct((_B, _OUT), jnp.float32)),
)


def kernel(input_ids, attention_mask, emb_g, W_g, b_g, emb_e, W_e, b_e,
           Wg1, bg1, g_gamma, g_beta, Wg2, bg2,
           Wc1, bc1, c_gamma, c_beta, Wc2, bc2):
    ids0 = input_ids[:, 0]
    m0 = attention_mask[:, 0].astype(jnp.float32).reshape(_B, 1)
    x_g, xe3 = _make_sc_gather()(ids0, emb_g, emb_e)

    out_c, out_p = _tc_dense(
        x_g, xe3, m0, W_g, b_g, W_e, b_e,
        Wg1, bg1, g_gamma, g_beta, Wg2, bg2,
        Wc1, bc1, c_gamma, c_beta, Wc2, bc2)
    return out_c, out_p


# speedup vs baseline: 1.0396x; 1.0008x over previous
import jax
import jax.numpy as jnp
from jax import lax
from jax.experimental import pallas as pl
from jax.experimental.pallas import tpu as pltpu, tpu_sc as plsc

_V = 30522
_D = 768
_H = 256
_B = 256
_S = 128
_E = 3
_OUT = 3
_NS = 16
_TG = _B // _NS


def _sc_gather_body(ids_hbm, emb_g_hbm, emb_e_hbm,
                    out_g_hbm, out_e_hbm,
                    blk_v, rows0_v, rows1_v, sem0, sem1):
    c = lax.axis_index("c")
    s = lax.axis_index("s")
    base = s * _TG
    pltpu.sync_copy(ids_hbm.at[pl.ds(base, _TG)], blk_v)

    @pl.when(c == 0)
    def _():
        cp0 = pltpu.async_copy(emb_g_hbm.at[blk_v], rows0_v, sem0)
        cp1 = pltpu.async_copy(emb_e_hbm.at[0].at[blk_v], rows1_v, sem1)
        cp0.wait()
        o0 = pltpu.async_copy(rows0_v, out_g_hbm.at[pl.ds(base, _TG)], sem0)
        cp1.wait()
        o1 = pltpu.async_copy(rows1_v, out_e_hbm.at[0, pl.ds(base, _TG)],
                              sem1)
        o0.wait()
        o1.wait()

    @pl.when(c == 1)
    def _():
        cp0 = pltpu.async_copy(emb_e_hbm.at[1].at[blk_v], rows0_v, sem0)
        cp1 = pltpu.async_copy(emb_e_hbm.at[2].at[blk_v], rows1_v, sem1)
        cp0.wait()
        o0 = pltpu.async_copy(rows0_v, out_e_hbm.at[1, pl.ds(base, _TG)],
                              sem0)
        cp1.wait()
        o1 = pltpu.async_copy(rows1_v, out_e_hbm.at[2, pl.ds(base, _TG)],
                              sem1)
        o0.wait()
        o1.wait()


def _make_sc_gather():
    return pl.kernel(
        _sc_gather_body,
        out_type=(jax.ShapeDtypeStruct((_B, _D), jnp.float32),
                  jax.ShapeDtypeStruct((_E, _B, _D), jnp.float32)),
        mesh=plsc.VectorSubcoreMesh(core_axis_name="c", subcore_axis_name="s"),
        scratch_types=[
            pltpu.VMEM((_TG,), jnp.int32),
            pltpu.VMEM((_TG, _D), jnp.float32),
            pltpu.VMEM((_TG, _D), jnp.float32),
            pltpu.SemaphoreType.DMA,
            pltpu.SemaphoreType.DMA,
        ],
    )


def _ln_rows(x, gamma, beta):
    mu = jnp.mean(x, axis=-1, keepdims=True)
    v = jnp.mean((x - mu) ** 2, axis=-1, keepdims=True)
    return (x - mu) * lax.rsqrt(v + 1e-5) * gamma + beta


def _tc_body(xg_ref, xe_ref, m0_ref, Wg_ref, bg_ref, We_ref, be_ref,
             Wg1_ref, bg1_ref, gga_ref, gbe_ref, Wg2_ref, bg2_ref,
             Wc1_ref, bc1_ref, cga_ref, cbe_ref, Wc2_ref, bc2_ref,
             out_c_ref, out_p_ref):
    f32 = jnp.float32
    m0 = m0_ref[...]
    xg = xg_ref[...] * m0
    h = jax.nn.gelu(jnp.dot(xg, Wg_ref[...], preferred_element_type=f32)
                    + bg_ref[...].reshape(1, _D))
    g = (jnp.dot(h, Wg1_ref[...], preferred_element_type=f32)
         + bg1_ref[...].reshape(1, _H))
    g = jax.nn.gelu(_ln_rows(g, gga_ref[...].reshape(1, _H),
                             gbe_ref[...].reshape(1, _H)))
    logits = (jnp.dot(g, Wg2_ref[...], preferred_element_type=f32)
              + bg2_ref[...].reshape(1, _OUT))
    lmax = jnp.max(logits, axis=-1, keepdims=True)
    e = jnp.exp(logits - lmax)
    p = e / jnp.sum(e, axis=-1, keepdims=True)
    out_p_ref[...] = p
    p0, p1, p2 = p[:, 0:1], p[:, 1:2], p[:, 2:3]
    excl2 = (p2 <= p0) & (p2 <= p1)
    excl1 = jnp.logical_not(excl2) & (p1 <= p0) & (p1 < p2)
    excl0 = jnp.logical_not(excl2) & jnp.logical_not(excl1)
    w0 = jnp.where(excl0, 0.0, p0)
    w1 = jnp.where(excl1, 0.0, p1)
    w2 = jnp.where(excl2, 0.0, p2)
    denom = w0 + w1 + w2
    ws = (w0 / denom, w1 / denom, w2 / denom)
    acc = jnp.zeros((_B, _D), f32)
    for i in range(_E):
        xe = xe_ref[i] * m0
        he = jax.nn.gelu(jnp.dot(xe, We_ref[i], preferred_element_type=f32)
                         + be_ref[i].reshape(1, _D))
        acc = acc + he * ws[i]
    c = (jnp.dot(acc, Wc1_ref[...], preferred_element_type=f32)
         + bc1_ref[...].reshape(1, _H))
    c = jax.nn.gelu(_ln_rows(c, cga_ref[...].reshape(1, _H),
                             cbe_ref[...].reshape(1, _H)))
    out_c_ref[...] = (jnp.dot(c, Wc2_ref[...], preferred_element_type=f32)
                      + bc2_ref[...].reshape(1, _OUT))


_tc_dense = pl.pallas_call(
    _tc_body,
    out_shape=(jax.ShapeDtypeStruct((_B, _OUT), jnp.float32),
               jax.ShapeDtypeStruct((_B, _OUT), jnp.float32)),
)


def kernel(input_ids, attention_mask, emb_g, W_g, b_g, emb_e, W_e, b_e,
           Wg1, bg1, g_gamma, g_beta, Wg2, bg2,
           Wc1, bc1, c_gamma, c_beta, Wc2, bc2):
    ids0 = input_ids[:, 0]
    m0 = attention_mask[:, 0].astype(jnp.float32).reshape(_B, 1)
    x_g, xe3 = _make_sc_gather()(ids0, emb_g, emb_e)

    out_c, out_p = _tc_dense(
        x_g, xe3, m0, W_g, b_g, W_e, b_e,
        Wg1, bg1, g_gamma, g_beta, Wg2, bg2,
        Wc1, bc1, c_gamma, c_beta, Wc2, bc2)
    return out_c, out_p
